# baseline (speedup 1.0000x reference)
import functools

import jax
import jax.numpy as jnp
from jax import lax
from jax.experimental import pallas as pl
from jax.experimental.pallas import tpu as pltpu
from jax.experimental.pallas import tpu_sc as plsc

_B = 4096
_N = 20
_D = 32
_P = _N * (_N - 1) // 2
_CHUNK = 128


def _sc_gather(mem_idx, user_idx, item_table, user_table):
    info = plsc.get_sparse_core_info()
    nw = info.num_cores * info.num_subcores
    rows = mem_idx.shape[0]
    urows = user_idx.shape[0]
    rows_per_w = rows // nw
    urows_per_w = urows // nw
    n_chunks = rows_per_w // _CHUNK
    un_chunks = urows_per_w // _CHUNK

    mesh = plsc.VectorSubcoreMesh(core_axis_name="c", subcore_axis_name="s")

    @functools.partial(
        pl.kernel,
        mesh=mesh,
        out_type=[
            jax.ShapeDtypeStruct((rows, _D), jnp.float32),
            jax.ShapeDtypeStruct((urows, _D), jnp.float32),
        ],
        scratch_types=[
            pltpu.VMEM((_CHUNK,), jnp.int32),
            pltpu.VMEM((_CHUNK, _D), jnp.float32),
            pltpu.SemaphoreType.DMA,
        ],
        compiler_params=pltpu.CompilerParams(use_tc_tiling_on_sc=False),
    )
    def k(mem_idx_hbm, user_idx_hbm, item_t_hbm, user_t_hbm,
          item_out, user_out, idx_v, rows_v, sem):
        wid = lax.axis_index("s") * info.num_cores + lax.axis_index("c")
        base = wid * rows_per_w
        for j in range(n_chunks):
            off = base + j * _CHUNK
            pltpu.sync_copy(mem_idx_hbm.at[pl.ds(off, _CHUNK)], idx_v)
            pltpu.async_copy(item_t_hbm.at[idx_v], rows_v, sem).wait()
            pltpu.sync_copy(rows_v, item_out.at[pl.ds(off, _CHUNK)])
        ubase = wid * urows_per_w
        for j in range(un_chunks):
            off = ubase + j * _CHUNK
            pltpu.sync_copy(user_idx_hbm.at[pl.ds(off, _CHUNK)], idx_v)
            pltpu.async_copy(user_t_hbm.at[idx_v], rows_v, sem).wait()
            pltpu.sync_copy(rows_v, user_out.at[pl.ds(off, _CHUNK)])

    return k(mem_idx, user_idx, item_table, user_table)


def _expand_body(uref, iref, wref, oref):
    ue = uref[...]
    we = iref[...] * wref[...]
    parts = [jnp.concatenate([ue] * _N, axis=1) * we]
    for i in range(_N - 1):
        li = we[:, i * _D:(i + 1) * _D]
        rep = _N - 1 - i
        parts.append(jnp.concatenate([li] * rep, axis=1) * we[:, (i + 1) * _D:])
    oref[...] = jnp.concatenate(parts, axis=1)


def kernel(user, memory, user_table, item_table, weights):
    user_idx = user.reshape(-1).astype(jnp.int32)
    mem_idx = memory.reshape(-1).astype(jnp.int32)

    item_emb, user_emb = _sc_gather(mem_idx, user_idx, item_table, user_table)
    item2d = item_emb.reshape(_B, _N * _D)
    wcols = jnp.repeat(weights, _D)[None, :]

    bb = 128
    grid = (_B // bb,)
    out = pl.pallas_call(
        _expand_body,
        grid=grid,
        in_specs=[
            pl.BlockSpec((bb, _D), lambda i: (i, 0)),
            pl.BlockSpec((bb, _N * _D), lambda i: (i, 0)),
            pl.BlockSpec((1, _N * _D), lambda i: (0, 0)),
        ],
        out_specs=pl.BlockSpec((bb, (_N + _P) * _D), lambda i: (i, 0)),
        out_shape=jax.ShapeDtypeStruct((_B, (_N + _P) * _D), jnp.float32),
    )(user_emb, item2d, wcols)
    return out
